# Initial kernel scaffold; baseline (speedup 1.0000x reference)
import jax
import jax.numpy as jnp
from jax.experimental import pallas as pl


def kernel(v, a, params):
    raise NotImplementedError("write your pallas kernel here")


# speedup vs baseline: 1.6876x; 1.6876x over previous
import jax
import jax.numpy as jnp
from jax.experimental import pallas as pl

_T = 20
_V = 32
_X = _T * _V
_BB = 128


def _prelu(x, a):
    return jnp.where(x >= 0.0, x, a * x)


def _c1x1(x, w, b2):
    y = jnp.einsum('bcx,oc->box', x, w, preferred_element_type=jnp.float32)
    return y + b2[0][None, :, None]


def _gcn(g, m):
    return jnp.einsum('box,xy->boy', g, m, preferred_element_type=jnp.float32)


def _tconv(y1, twf, tb2):
    ypad = jnp.pad(y1, ((0, 0), (0, 0), (_V, _V)))
    ycat = jnp.concatenate(
        [ypad[:, :, k * _V:k * _V + _X] for k in range(3)], axis=1)
    y = jnp.einsum('bix,oi->box', ycat, twf, preferred_element_type=jnp.float32)
    return y + tb2[0][None, :, None]


def _stats_body(x_ref, m_ref, gw_ref, gb_ref, rw_ref, rb_ref, acc_ref):
    i = pl.program_id(0)
    x = x_ref[...]
    y0 = _gcn(_c1x1(x, gw_ref[...], gb_ref[...]), m_ref[...])
    res = _c1x1(x, rw_ref[...], rb_ref[...])
    s = jnp.concatenate([
        jnp.sum(y0, axis=(0, 2))[None, :],
        jnp.sum(y0 * y0, axis=(0, 2))[None, :],
        jnp.sum(res, axis=(0, 2))[None, :],
        jnp.sum(res * res, axis=(0, 2))[None, :],
    ], axis=0)

    @pl.when(i == 0)
    def _():
        acc_ref[...] = s

    @pl.when(i != 0)
    def _():
        acc_ref[...] += s


def _mid_body(x_ref, m_ref, gw_ref, gb_ref, sc0_ref, sh0_ref, a1_ref,
              twf_ref, tb_ref, y2_ref, acc_ref):
    i = pl.program_id(0)
    x = x_ref[...]
    y0 = _gcn(_c1x1(x, gw_ref[...], gb_ref[...]), m_ref[...])
    y1 = _prelu(y0 * sc0_ref[0][None, :, None] + sh0_ref[0][None, :, None],
                a1_ref[0, 0])
    y2 = _tconv(y1, twf_ref[...], tb_ref[...])
    y2_ref[...] = y2
    s = jnp.concatenate([
        jnp.sum(y2, axis=(0, 2))[None, :],
        jnp.sum(y2 * y2, axis=(0, 2))[None, :],
    ], axis=0)

    @pl.when(i == 0)
    def _():
        acc_ref[...] = s

    @pl.when(i != 0)
    def _():
        acc_ref[...] += s


def _out_body(y2_ref, x_ref, rw_ref, rb_ref, sc2_ref, sh2_ref,
              scr_ref, shr_ref, a2_ref, out_ref):
    res = _c1x1(x_ref[...], rw_ref[...], rb_ref[...])
    t = (y2_ref[...] * sc2_ref[0][None, :, None] + sh2_ref[0][None, :, None]
         + res * scr_ref[0][None, :, None] + shr_ref[0][None, :, None])
    out_ref[...] = _prelu(t, a2_ref[0, 0])


def _full(arr):
    nd = arr.ndim
    return pl.BlockSpec(arr.shape, lambda i, _nd=nd: (0,) * _nd)


def _bspec(c):
    return pl.BlockSpec((_BB, c, _X), lambda i: (i, 0, 0))


def _bn_coeffs(s, s2, g, b, count, eps=1e-5):
    mean = s / count
    var = s2 / count - mean * mean
    sc = g / jnp.sqrt(var + eps)
    return (sc[None, :], (b - mean * sc)[None, :])


def _run_block(x, m, p):
    n, cin = x.shape[0], x.shape[1]
    cout = p['gw'].shape[0]
    grid = (n // _BB,)
    count = jnp.float32(n * _X)
    f32 = jnp.float32

    gb2 = p['gb'][None, :]
    rb2 = p['rb'][None, :]
    tb2 = p['tb'][None, :]
    twf = jnp.concatenate([p['tw'][:, :, k, 0] for k in range(3)], axis=1)
    a1 = jnp.asarray(p['a1'], f32).reshape(1, 1)
    a2 = jnp.asarray(p['a2'], f32).reshape(1, 1)

    stats = pl.pallas_call(
        _stats_body,
        grid=grid,
        in_specs=[_bspec(cin)] + [_full(z) for z in
                  (m, p['gw'], gb2, p['rw'], rb2)],
        out_specs=pl.BlockSpec((4, cout), lambda i: (0, 0)),
        out_shape=jax.ShapeDtypeStruct((4, cout), f32),
    )(x, m, p['gw'], gb2, p['rw'], rb2)

    sc0, sh0 = _bn_coeffs(stats[0], stats[1], p['g1'], p['b1'], count)
    scr, shr = _bn_coeffs(stats[2], stats[3], p['rg'], p['rbb'], count)

    y2, s2 = pl.pallas_call(
        _mid_body,
        grid=grid,
        in_specs=[_bspec(cin)] + [_full(z) for z in
                  (m, p['gw'], gb2, sc0, sh0, a1, twf, tb2)],
        out_specs=[_bspec(cout), pl.BlockSpec((2, cout), lambda i: (0, 0))],
        out_shape=[jax.ShapeDtypeStruct((n, cout, _X), f32),
                   jax.ShapeDtypeStruct((2, cout), f32)],
    )(x, m, p['gw'], gb2, sc0, sh0, a1, twf, tb2)

    sc2, sh2 = _bn_coeffs(s2[0], s2[1], p['g2'], p['b2'], count)

    out = pl.pallas_call(
        _out_body,
        grid=grid,
        in_specs=[_bspec(cout), _bspec(cin)] + [_full(z) for z in
                  (p['rw'], rb2, sc2, sh2, scr, shr, a2)],
        out_specs=_bspec(cout),
        out_shape=jax.ShapeDtypeStruct((n, cout, _X), f32),
    )(y2, x, p['rw'], rb2, sc2, sh2, scr, shr, a2)
    return out


def kernel(v, a, params):
    n = v.shape[0]
    t, vv = a.shape[0], a.shape[1]
    x = v.reshape(n, v.shape[1], _X)
    m4 = a[:, :, None, :] * jnp.eye(t, dtype=a.dtype)[:, None, :, None]
    m = m4.reshape(_X, _X)

    x = _run_block(x, m, params['e1'])
    ev = _run_block(x, m, params['e2'])
    x = _run_block(ev, m, params['d1'])
    dv = _run_block(x, m, params['d2'])

    ev = ev.reshape(n, ev.shape[1], t, vv)
    dv = dv.reshape(n, dv.shape[1], t, vv)
    return (ev, dv)


# speedup vs baseline: 1.8767x; 1.1120x over previous
import jax
import jax.numpy as jnp
from jax.experimental import pallas as pl
from jax.experimental.pallas import tpu as pltpu

_T = 20
_V = 32
_X = _T * _V
_G = 128
_NG = _X // _G
_BB = 128


def _prelu(x, a):
    return jnp.where(x >= 0.0, x, a * x)


def _c1x1(x, w, b2):
    y = jnp.einsum('bcx,oc->box', x, w, preferred_element_type=jnp.float32)
    return y + b2[0][None, :, None]


def _gcn(g, m):
    parts = [jnp.einsum('box,xy->boy', g[:, :, i * _G:(i + 1) * _G], m[i],
                        preferred_element_type=jnp.float32)
             for i in range(_NG)]
    return jnp.concatenate(parts, axis=2)


def _tconv(y1, twf, tb2):
    ypad = jnp.pad(y1, ((0, 0), (0, 0), (_V, _V)))
    ycat = jnp.concatenate(
        [ypad[:, :, k * _V:k * _V + _X] for k in range(3)], axis=1)
    y = jnp.einsum('bix,oi->box', ycat, twf, preferred_element_type=jnp.float32)
    return y + tb2[0][None, :, None]


def _stats_body(x_ref, m_ref, gw_ref, gb_ref, rw_ref, rb_ref, acc_ref):
    i = pl.program_id(0)
    x = x_ref[...]
    y0 = _gcn(_c1x1(x, gw_ref[...], gb_ref[...]), m_ref[...])
    res = _c1x1(x, rw_ref[...], rb_ref[...])
    del i
    acc_ref[0] = jnp.concatenate([
        jnp.sum(y0, axis=(0, 2))[None, :],
        jnp.sum(y0 * y0, axis=(0, 2))[None, :],
        jnp.sum(res, axis=(0, 2))[None, :],
        jnp.sum(res * res, axis=(0, 2))[None, :],
    ], axis=0)


def _mid_body(x_ref, m_ref, gw_ref, gb_ref, sc0_ref, sh0_ref, a1_ref,
              twf_ref, tb_ref, y2_ref, acc_ref):
    i = pl.program_id(0)
    x = x_ref[...]
    y0 = _gcn(_c1x1(x, gw_ref[...], gb_ref[...]), m_ref[...])
    y1 = _prelu(y0 * sc0_ref[0][None, :, None] + sh0_ref[0][None, :, None],
                a1_ref[0, 0])
    y2 = _tconv(y1, twf_ref[...], tb_ref[...])
    del i
    y2_ref[...] = y2
    acc_ref[0] = jnp.concatenate([
        jnp.sum(y2, axis=(0, 2))[None, :],
        jnp.sum(y2 * y2, axis=(0, 2))[None, :],
    ], axis=0)


def _out_body(y2_ref, x_ref, rw_ref, rb_ref, sc2_ref, sh2_ref,
              scr_ref, shr_ref, a2_ref, out_ref):
    res = _c1x1(x_ref[...], rw_ref[...], rb_ref[...])
    t = (y2_ref[...] * sc2_ref[0][None, :, None] + sh2_ref[0][None, :, None]
         + res * scr_ref[0][None, :, None] + shr_ref[0][None, :, None])
    out_ref[...] = _prelu(t, a2_ref[0, 0])


def _full(arr):
    nd = arr.ndim
    return pl.BlockSpec(arr.shape, lambda i, _nd=nd: (0,) * _nd)


def _bspec(c, bb=_BB):
    return pl.BlockSpec((bb, c, _X), lambda i: (i, 0, 0))


def _bn_coeffs(s, s2, g, b, count, eps=1e-5):
    mean = s / count
    var = s2 / count - mean * mean
    sc = g / jnp.sqrt(var + eps)
    return (sc[None, :], (b - mean * sc)[None, :])


def _run_block(x, m, p):
    n, cin = x.shape[0], x.shape[1]
    cout = p['gw'].shape[0]
    grid = (n // _BB,)
    count = jnp.float32(n * _X)
    f32 = jnp.float32

    gb2 = p['gb'][None, :]
    rb2 = p['rb'][None, :]
    tb2 = p['tb'][None, :]
    twf = jnp.concatenate([p['tw'][:, :, k, 0] for k in range(3)], axis=1)
    a1 = jnp.asarray(p['a1'], f32).reshape(1, 1)
    a2 = jnp.asarray(p['a2'], f32).reshape(1, 1)

    par = pltpu.CompilerParams(dimension_semantics=("parallel",))
    nsteps = grid[0]

    stats = pl.pallas_call(
        _stats_body,
        grid=grid,
        in_specs=[_bspec(cin)] + [_full(z) for z in
                  (m, p['gw'], gb2, p['rw'], rb2)],
        out_specs=pl.BlockSpec((1, 4, cout), lambda i: (i, 0, 0)),
        out_shape=jax.ShapeDtypeStruct((nsteps, 4, cout), f32),
        compiler_params=par,
    )(x, m, p['gw'], gb2, p['rw'], rb2).sum(axis=0)

    sc0, sh0 = _bn_coeffs(stats[0], stats[1], p['g1'], p['b1'], count)
    scr, shr = _bn_coeffs(stats[2], stats[3], p['rg'], p['rbb'], count)

    bbm = _BB // 2
    y2, s2 = pl.pallas_call(
        _mid_body,
        grid=(n // bbm,),
        in_specs=[_bspec(cin, bbm)] + [_full(z) for z in
                  (m, p['gw'], gb2, sc0, sh0, a1, twf, tb2)],
        out_specs=[_bspec(cout, bbm),
                   pl.BlockSpec((1, 2, cout), lambda i: (i, 0, 0))],
        out_shape=[jax.ShapeDtypeStruct((n, cout, _X), f32),
                   jax.ShapeDtypeStruct((n // bbm, 2, cout), f32)],
        compiler_params=par,
    )(x, m, p['gw'], gb2, sc0, sh0, a1, twf, tb2)
    s2 = s2.sum(axis=0)

    sc2, sh2 = _bn_coeffs(s2[0], s2[1], p['g2'], p['b2'], count)

    out = pl.pallas_call(
        _out_body,
        grid=grid,
        in_specs=[_bspec(cout), _bspec(cin)] + [_full(z) for z in
                  (p['rw'], rb2, sc2, sh2, scr, shr, a2)],
        out_specs=_bspec(cout),
        out_shape=jax.ShapeDtypeStruct((n, cout, _X), f32),
        compiler_params=par,
    )(y2, x, p['rw'], rb2, sc2, sh2, scr, shr, a2)
    return out


def kernel(v, a, params):
    n = v.shape[0]
    t, vv = a.shape[0], a.shape[1]
    x = v.reshape(n, v.shape[1], _X)
    m4 = a[:, :, None, :] * jnp.eye(t, dtype=a.dtype)[:, None, :, None]
    mfull = m4.reshape(_X, _X)
    m = jnp.stack([mfull[i * _G:(i + 1) * _G, i * _G:(i + 1) * _G]
                   for i in range(_NG)])

    x = _run_block(x, m, params['e1'])
    ev = _run_block(x, m, params['e2'])
    x = _run_block(ev, m, params['d1'])
    dv = _run_block(x, m, params['d2'])

    ev = ev.reshape(n, ev.shape[1], t, vv)
    dv = dv.reshape(n, dv.shape[1], t, vv)
    return (ev, dv)
